# baseline (speedup 1.0000x reference)
import jax
import jax.numpy as jnp
from jax.experimental import pallas as pl

H, W, C = 512, 512, 63
CB = 9


def _tc_body(img_ref, out_ref):
    c0 = pl.program_id(0) * CB
    cls = jax.lax.broadcasted_iota(jnp.int32, (CB, 1, 1), 0) + (c0 + 1)
    out_ref[...] = (img_ref[...] == cls).astype(jnp.int32)


@jax.jit
def _onehot(img):
    enc = pl.pallas_call(
        _tc_body,
        out_shape=jax.ShapeDtypeStruct((C, H, W), jnp.int32),
        grid=(C // CB,),
        in_specs=[pl.BlockSpec((1, H, W), lambda c: (0, 0, 0))],
        out_specs=pl.BlockSpec((CB, H, W), lambda c: (c, 0, 0)),
    )(img)
    return enc.transpose(1, 2, 0)


def kernel(img):
    return _onehot(img)
